# baseline (speedup 1.0000x reference)
import jax
import jax.numpy as jnp
from jax import lax
from jax.experimental import pallas as pl
from jax.experimental.pallas import tpu as pltpu
from jax.experimental.pallas import tpu_sc as plsc

_N = 100000
_NW = 16
_CHUNK = 6240
_VREGS = _CHUNK // 16
_TAIL_BASE = _NW * _CHUNK
_TAIL_VREGS = (_N - _TAIL_BASE) // 16

_F32 = jnp.float32
_I32 = jnp.int32


def _sqrt_f32(v):
    bits = plsc.bitcast(v, _I32)
    seed = plsc.bitcast(jnp.int32(0x5F3759DF) - lax.shift_right_logical(bits, 1), _F32)
    y = seed
    half = v * 0.5
    for _ in range(3):
        y = y * (1.5 - half * y * y)
    return v * y


_UNROLL = 13
assert _VREGS % _UNROLL == 0


def _idloss_body(pred_hbm, tgt_hbm, out_hbm,
                 predv, tgtv, xtrx, xtrt,
                 acc_s, acc_q, acc_c, combv, stage, outv, shared,
                 semA, semB, semT):
    wid = lax.axis_index("s")
    lane = lax.iota(_I32, 16)
    lane_base = lane * 16
    zeros16 = jnp.zeros((16,), _F32)
    ones16 = jnp.ones((16,), _F32)

    base = wid * _CHUNK
    half = _CHUNK // 2
    toff = _TAIL_BASE + wid * 16
    cpa1 = pltpu.async_copy(pred_hbm.at[pl.ds(base, half)],
                            predv.at[pl.ds(0, half)], semA)
    cpa2 = pltpu.async_copy(tgt_hbm.at[pl.ds(base, half)],
                            tgtv.at[pl.ds(0, half)], semA)
    cpb1 = pltpu.async_copy(pred_hbm.at[pl.ds(base + half, half)],
                            predv.at[pl.ds(half, half)], semB)
    cpb2 = pltpu.async_copy(tgt_hbm.at[pl.ds(base + half, half)],
                            tgtv.at[pl.ds(half, half)], semB)

    @pl.when(wid < _TAIL_VREGS)
    def _():
        pltpu.async_copy(pred_hbm.at[pl.ds(toff, 16)], xtrx, semT)
        pltpu.async_copy(tgt_hbm.at[pl.ds(toff, 16)], xtrt, semT)

    for b in range(16):
        acc_s[pl.ds(b * 16, 16)] = zeros16
        acc_q[pl.ds(b * 16, 16)] = zeros16
        acc_c[pl.ds(b * 16, 16)] = zeros16

    def scat(x, t):
        idx = lane_base + t
        plsc.addupdate_scatter(acc_s, [idx], x)
        plsc.addupdate_scatter(acc_q, [idx], x * x)
        plsc.addupdate_scatter(acc_c, [idx], ones16)

    cpa1.wait()
    cpa2.wait()

    @plsc.parallel_loop(0, _VREGS // 2, 1, unroll=_UNROLL)
    def _(k):
        o = k * 16
        scat(predv[pl.ds(o, 16)], tgtv[pl.ds(o, 16)])

    cpb1.wait()
    cpb2.wait()

    @plsc.parallel_loop(_VREGS // 2, _VREGS, 1, unroll=_UNROLL)
    def _(k):
        o = k * 16
        scat(predv[pl.ds(o, 16)], tgtv[pl.ds(o, 16)])

    @pl.when(wid < _TAIL_VREGS)
    def _():
        pltpu.make_async_copy(pred_hbm.at[pl.ds(toff, 16)], xtrx, semT).wait()
        pltpu.make_async_copy(tgt_hbm.at[pl.ds(toff, 16)], xtrt, semT).wait()
        scat(xtrx[...], xtrt[...])

    s16 = zeros16
    q16 = zeros16
    c16 = zeros16
    for l in range(16):
        s16 = s16 + acc_s[pl.ds(l * 16, 16)]
        q16 = q16 + acc_q[pl.ds(l * 16, 16)]
        c16 = c16 + acc_c[pl.ds(l * 16, 16)]
    stage[pl.ds(0, 16)] = s16
    stage[pl.ds(16, 16)] = q16
    stage[pl.ds(32, 16)] = c16
    pltpu.sync_copy(stage, shared.at[pl.ds(wid * 48, 48)])
    plsc.subcore_barrier()

    @pl.when(wid == 0)
    def _():
        pltpu.sync_copy(shared, combv)

        S = zeros16
        Q = zeros16
        C = zeros16
        for w in range(16):
            S = S + combv[pl.ds(w * 48, 16)]
            Q = Q + combv[pl.ds(w * 48 + 16, 16)]
            C = C + combv[pl.ds(w * 48 + 32, 16)]

        M = S / C
        SSQ = Q - C * M * M
        SSQ = jnp.where(SSQ < 0.0, zeros16, SSQ)
        VAR = SSQ / (C - 1.0)
        STD = _sqrt_f32(VAR)

        s_nan = STD != STD
        c_pos = C > 0.0

        def pair_body(i, carry):
            tot, cnt = carry
            bidx = jnp.full((16,), i, _I32)
            pi = M.at[bidx].get(mode="promise_in_bounds")
            si = STD.at[bidx].get(mode="promise_in_bounds")
            ci = C.at[bidx].get(mode="promise_in_bounds")
            vd = M - pi
            dn = jnp.abs(vd)
            vdir = vd / dn
            ii_ = jnp.where(si != si, zeros16, jnp.abs(si * vdir))
            jj_ = jnp.where(s_nan, zeros16, jnp.abs(STD * vdir))
            viou = (ii_ + jj_) / (dn + ii_ + jj_)
            valid = jnp.logical_and(jnp.logical_and(ci > 0.0, c_pos), lane > i)
            tot = tot + jnp.where(valid, viou, zeros16)
            cnt = cnt + jnp.where(valid, ones16, zeros16)
            return tot, cnt

        tot, cnt = lax.fori_loop(0, 16, pair_body, (zeros16, zeros16))

        vd16 = 0.0 - M
        dn16 = jnp.abs(vd16)
        vdir16 = vd16 / dn16
        ii16 = jnp.where(s_nan, zeros16, jnp.abs(STD * vdir16))
        jj16 = jnp.abs(0.0 * vdir16)
        viou16 = (ii16 + jj16) / (dn16 + ii16 + jj16)
        tot = tot + jnp.where(c_pos, viou16, zeros16)
        cnt = cnt + jnp.where(c_pos, ones16, zeros16)

        tot_v = jnp.full((16,), jnp.sum(tot), _F32)
        cnt_v = jnp.full((16,), jnp.sum(cnt), _F32)
        outv[...] = tot_v / cnt_v
        pltpu.sync_copy(outv.at[pl.ds(0, 1)], out_hbm)


@jax.jit
def _idloss(pred_id, target_id):
    mesh = plsc.VectorSubcoreMesh(core_axis_name="c", subcore_axis_name="s",
                                  num_cores=1)
    out = pl.kernel(
        _idloss_body,
        mesh=mesh,
        compiler_params=pltpu.CompilerParams(
            needs_layout_passes=False,
            disable_bounds_checks=True,
            disable_semaphore_checks=True,
            skip_device_barrier=True,
        ),
        out_type=jax.ShapeDtypeStruct((1,), _F32),
        scratch_types=[
            pltpu.VMEM((_CHUNK,), _F32),
            pltpu.VMEM((_CHUNK,), _I32),
            pltpu.VMEM((16,), _F32),
            pltpu.VMEM((16,), _I32),
            pltpu.VMEM((256,), _F32),
            pltpu.VMEM((256,), _F32),
            pltpu.VMEM((256,), _F32),
            pltpu.VMEM((768,), _F32),
            pltpu.VMEM((48,), _F32),
            pltpu.VMEM((16,), _F32),
            pltpu.VMEM_SHARED((768,), _F32),
            pltpu.SemaphoreType.DMA,
            pltpu.SemaphoreType.DMA,
            pltpu.SemaphoreType.DMA,
        ],
    )(pred_id, target_id)
    return jnp.reshape(out, ())


def kernel(pred_id, target_id):
    return _idloss(pred_id, target_id)


# speedup vs baseline: 1.0026x; 1.0026x over previous
import jax
import jax.numpy as jnp
from jax import lax
from jax.experimental import pallas as pl
from jax.experimental.pallas import tpu as pltpu
from jax.experimental.pallas import tpu_sc as plsc

_N = 100000
_NW = 16
_CHUNK = 6240
_VREGS = _CHUNK // 16
_TAIL_BASE = _NW * _CHUNK
_TAIL_VREGS = (_N - _TAIL_BASE) // 16

_F32 = jnp.float32
_I32 = jnp.int32


def _sqrt_f32(v):
    bits = plsc.bitcast(v, _I32)
    seed = plsc.bitcast(jnp.int32(0x5F3759DF) - lax.shift_right_logical(bits, 1), _F32)
    y = seed
    half = v * 0.5
    for _ in range(3):
        y = y * (1.5 - half * y * y)
    return v * y


_UNROLL = 13
assert _VREGS % _UNROLL == 0


def _idloss_body(pred_hbm, tgt_hbm, out_hbm,
                 predv, tgtv, xtrx, xtrt,
                 acc_s, acc_q, acc_c, combv, stage, outv, shared,
                 semA, semB, semT):
    wid = lax.axis_index("s")
    lane = lax.iota(_I32, 16)
    lane_base = lane * 16
    zeros16 = jnp.zeros((16,), _F32)
    ones16 = jnp.ones((16,), _F32)

    base = wid * _CHUNK
    half = _CHUNK // 2
    toff = _TAIL_BASE + wid * 16
    cpa1 = pltpu.async_copy(pred_hbm.at[pl.ds(base, half)],
                            predv.at[pl.ds(0, half)], semA)
    cpa2 = pltpu.async_copy(tgt_hbm.at[pl.ds(base, half)],
                            tgtv.at[pl.ds(0, half)], semA)
    cpb1 = pltpu.async_copy(pred_hbm.at[pl.ds(base + half, half)],
                            predv.at[pl.ds(half, half)], semB)
    cpb2 = pltpu.async_copy(tgt_hbm.at[pl.ds(base + half, half)],
                            tgtv.at[pl.ds(half, half)], semB)

    @pl.when(wid < _TAIL_VREGS)
    def _():
        pltpu.async_copy(pred_hbm.at[pl.ds(toff, 16)], xtrx, semT)
        pltpu.async_copy(tgt_hbm.at[pl.ds(toff, 16)], xtrt, semT)

    for b in range(16):
        acc_s[pl.ds(b * 16, 16)] = zeros16
        acc_q[pl.ds(b * 16, 16)] = zeros16
        acc_c[pl.ds(b * 16, 16)] = zeros16

    def scat(x, t):
        idx = lane_base + t
        plsc.addupdate_scatter(acc_s, [idx], x)
        plsc.addupdate_scatter(acc_q, [idx], x * x)
        plsc.addupdate_scatter(acc_c, [idx], ones16)

    cpa1.wait()
    cpa2.wait()

    @plsc.parallel_loop(0, _VREGS // 2, 1, unroll=_UNROLL)
    def _(k):
        o = k * 16
        scat(predv[pl.ds(o, 16)], tgtv[pl.ds(o, 16)])

    cpb1.wait()
    cpb2.wait()

    @plsc.parallel_loop(_VREGS // 2, _VREGS, 1, unroll=_UNROLL)
    def _(k):
        o = k * 16
        scat(predv[pl.ds(o, 16)], tgtv[pl.ds(o, 16)])

    @pl.when(wid < _TAIL_VREGS)
    def _():
        pltpu.make_async_copy(pred_hbm.at[pl.ds(toff, 16)], xtrx, semT).wait()
        pltpu.make_async_copy(tgt_hbm.at[pl.ds(toff, 16)], xtrt, semT).wait()
        scat(xtrx[...], xtrt[...])

    s16 = zeros16
    q16 = zeros16
    c16 = zeros16
    for l in range(16):
        s16 = s16 + acc_s[pl.ds(l * 16, 16)]
        q16 = q16 + acc_q[pl.ds(l * 16, 16)]
        c16 = c16 + acc_c[pl.ds(l * 16, 16)]
    stage[pl.ds(0, 16)] = s16
    stage[pl.ds(16, 16)] = q16
    stage[pl.ds(32, 16)] = c16
    pltpu.sync_copy(stage, shared.at[pl.ds(wid * 48, 48)])
    plsc.subcore_barrier()

    @pl.when(wid == 0)
    def _():
        pltpu.sync_copy(shared, combv)

        S = zeros16
        Q = zeros16
        C = zeros16
        for w in range(16):
            S = S + combv[pl.ds(w * 48, 16)]
            Q = Q + combv[pl.ds(w * 48 + 16, 16)]
            C = C + combv[pl.ds(w * 48 + 32, 16)]

        M = S / C
        SSQ = Q - C * M * M
        SSQ = jnp.where(SSQ < 0.0, zeros16, SSQ)
        VAR = SSQ / (C - 1.0)
        STD = _sqrt_f32(VAR)

        s_nan = STD != STD
        c_pos = C > 0.0

        def pair_one(i, tot, cnt):
            bidx = jnp.full((16,), i, _I32)
            pi = M.at[bidx].get(mode="promise_in_bounds")
            si = STD.at[bidx].get(mode="promise_in_bounds")
            ci = C.at[bidx].get(mode="promise_in_bounds")
            vd = M - pi
            dn = jnp.abs(vd)
            vdir = vd / dn
            ii_ = jnp.where(si != si, zeros16, jnp.abs(si * vdir))
            jj_ = jnp.where(s_nan, zeros16, jnp.abs(STD * vdir))
            viou = (ii_ + jj_) / (dn + ii_ + jj_)
            valid = jnp.logical_and(jnp.logical_and(ci > 0.0, c_pos), lane > i)
            tot = tot + jnp.where(valid, viou, zeros16)
            cnt = cnt + jnp.where(valid, ones16, zeros16)
            return tot, cnt

        def pair_body(g, carry):
            tot, cnt = carry
            for u in range(4):
                tot, cnt = pair_one(g * 4 + u, tot, cnt)
            return tot, cnt

        tot, cnt = lax.fori_loop(0, 4, pair_body, (zeros16, zeros16))

        vd16 = 0.0 - M
        dn16 = jnp.abs(vd16)
        vdir16 = vd16 / dn16
        ii16 = jnp.where(s_nan, zeros16, jnp.abs(STD * vdir16))
        jj16 = jnp.abs(0.0 * vdir16)
        viou16 = (ii16 + jj16) / (dn16 + ii16 + jj16)
        tot = tot + jnp.where(c_pos, viou16, zeros16)
        cnt = cnt + jnp.where(c_pos, ones16, zeros16)

        tot_v = jnp.full((16,), jnp.sum(tot), _F32)
        cnt_v = jnp.full((16,), jnp.sum(cnt), _F32)
        outv[...] = tot_v / cnt_v
        pltpu.sync_copy(outv.at[pl.ds(0, 1)], out_hbm)


@jax.jit
def _idloss(pred_id, target_id):
    mesh = plsc.VectorSubcoreMesh(core_axis_name="c", subcore_axis_name="s",
                                  num_cores=1)
    out = pl.kernel(
        _idloss_body,
        mesh=mesh,
        compiler_params=pltpu.CompilerParams(needs_layout_passes=False),
        out_type=jax.ShapeDtypeStruct((1,), _F32),
        scratch_types=[
            pltpu.VMEM((_CHUNK,), _F32),
            pltpu.VMEM((_CHUNK,), _I32),
            pltpu.VMEM((16,), _F32),
            pltpu.VMEM((16,), _I32),
            pltpu.VMEM((256,), _F32),
            pltpu.VMEM((256,), _F32),
            pltpu.VMEM((256,), _F32),
            pltpu.VMEM((768,), _F32),
            pltpu.VMEM((48,), _F32),
            pltpu.VMEM((16,), _F32),
            pltpu.VMEM_SHARED((768,), _F32),
            pltpu.SemaphoreType.DMA,
            pltpu.SemaphoreType.DMA,
            pltpu.SemaphoreType.DMA,
        ],
    )(pred_id, target_id)
    return jnp.reshape(out, ())


def kernel(pred_id, target_id):
    return _idloss(pred_id, target_id)


# speedup vs baseline: 1.0130x; 1.0104x over previous
import jax
import jax.numpy as jnp
from jax import lax
from jax.experimental import pallas as pl
from jax.experimental.pallas import tpu as pltpu
from jax.experimental.pallas import tpu_sc as plsc

_N = 100000
_NW = 16
_CHUNK = 6240
_VREGS = _CHUNK // 16
_TAIL_BASE = _NW * _CHUNK
_TAIL_VREGS = (_N - _TAIL_BASE) // 16

_F32 = jnp.float32
_I32 = jnp.int32


def _sqrt_f32(v):
    bits = plsc.bitcast(v, _I32)
    seed = plsc.bitcast(jnp.int32(0x5F3759DF) - lax.shift_right_logical(bits, 1), _F32)
    y = seed
    half = v * 0.5
    for _ in range(3):
        y = y * (1.5 - half * y * y)
    return v * y


_UNROLL = 13
assert _VREGS % _UNROLL == 0


def _idloss_body(pred_hbm, tgt_hbm, out_hbm,
                 predv, tgtv, xtrx, xtrt,
                 acc_s, acc_q, acc_c, combv, stage, outv, shared,
                 semA, semB, semT):
    wid = lax.axis_index("s")
    lane = lax.iota(_I32, 16)
    lane_base = lane * 16
    zeros16 = jnp.zeros((16,), _F32)
    ones16 = jnp.ones((16,), _F32)

    base = wid * _CHUNK
    half = _CHUNK // 2
    toff = _TAIL_BASE + wid * 16
    cpa1 = pltpu.async_copy(pred_hbm.at[pl.ds(base, half)],
                            predv.at[pl.ds(0, half)], semA)
    cpa2 = pltpu.async_copy(tgt_hbm.at[pl.ds(base, half)],
                            tgtv.at[pl.ds(0, half)], semA)

    @pl.when(wid < _TAIL_VREGS)
    def _():
        pltpu.async_copy(pred_hbm.at[pl.ds(toff, 16)], xtrx, semT)
        pltpu.async_copy(tgt_hbm.at[pl.ds(toff, 16)], xtrt, semT)

    for b in range(16):
        acc_s[pl.ds(b * 16, 16)] = zeros16
        acc_q[pl.ds(b * 16, 16)] = zeros16
        acc_c[pl.ds(b * 16, 16)] = zeros16

    def scat(x, t):
        idx = lane_base + t
        plsc.addupdate_scatter(acc_s, [idx], x)
        plsc.addupdate_scatter(acc_q, [idx], x * x)
        plsc.addupdate_scatter(acc_c, [idx], ones16)

    cpa1.wait()
    cpa2.wait()

    cpb1 = pltpu.async_copy(pred_hbm.at[pl.ds(base + half, half)],
                            predv.at[pl.ds(half, half)], semB)
    cpb2 = pltpu.async_copy(tgt_hbm.at[pl.ds(base + half, half)],
                            tgtv.at[pl.ds(half, half)], semB)

    @plsc.parallel_loop(0, _VREGS // 2, 1, unroll=_UNROLL)
    def _(k):
        o = k * 16
        scat(predv[pl.ds(o, 16)], tgtv[pl.ds(o, 16)])

    cpb1.wait()
    cpb2.wait()

    @plsc.parallel_loop(_VREGS // 2, _VREGS, 1, unroll=_UNROLL)
    def _(k):
        o = k * 16
        scat(predv[pl.ds(o, 16)], tgtv[pl.ds(o, 16)])

    @pl.when(wid < _TAIL_VREGS)
    def _():
        pltpu.make_async_copy(pred_hbm.at[pl.ds(toff, 16)], xtrx, semT).wait()
        pltpu.make_async_copy(tgt_hbm.at[pl.ds(toff, 16)], xtrt, semT).wait()
        scat(xtrx[...], xtrt[...])

    s16 = zeros16
    q16 = zeros16
    c16 = zeros16
    for l in range(16):
        s16 = s16 + acc_s[pl.ds(l * 16, 16)]
        q16 = q16 + acc_q[pl.ds(l * 16, 16)]
        c16 = c16 + acc_c[pl.ds(l * 16, 16)]
    stage[pl.ds(0, 16)] = s16
    stage[pl.ds(16, 16)] = q16
    stage[pl.ds(32, 16)] = c16
    pltpu.sync_copy(stage, shared.at[pl.ds(wid * 48, 48)])
    plsc.subcore_barrier()

    @pl.when(wid == 0)
    def _():
        pltpu.sync_copy(shared, combv)

        S = zeros16
        Q = zeros16
        C = zeros16
        for w in range(16):
            S = S + combv[pl.ds(w * 48, 16)]
            Q = Q + combv[pl.ds(w * 48 + 16, 16)]
            C = C + combv[pl.ds(w * 48 + 32, 16)]

        M = S / C
        SSQ = Q - C * M * M
        SSQ = jnp.where(SSQ < 0.0, zeros16, SSQ)
        VAR = SSQ / (C - 1.0)
        STD = _sqrt_f32(VAR)

        s_nan = STD != STD
        c_pos = C > 0.0

        def pair_one(i, tot, cnt):
            bidx = jnp.full((16,), i, _I32)
            pi = M.at[bidx].get(mode="promise_in_bounds")
            si = STD.at[bidx].get(mode="promise_in_bounds")
            ci = C.at[bidx].get(mode="promise_in_bounds")
            vd = M - pi
            dn = jnp.abs(vd)
            vdir = vd / dn
            ii_ = jnp.where(si != si, zeros16, jnp.abs(si * vdir))
            jj_ = jnp.where(s_nan, zeros16, jnp.abs(STD * vdir))
            viou = (ii_ + jj_) / (dn + ii_ + jj_)
            valid = jnp.logical_and(jnp.logical_and(ci > 0.0, c_pos), lane > i)
            tot = tot + jnp.where(valid, viou, zeros16)
            cnt = cnt + jnp.where(valid, ones16, zeros16)
            return tot, cnt

        def pair_body(g, carry):
            tot, cnt = carry
            for u in range(4):
                tot, cnt = pair_one(g * 4 + u, tot, cnt)
            return tot, cnt

        tot, cnt = lax.fori_loop(0, 4, pair_body, (zeros16, zeros16))

        vd16 = 0.0 - M
        dn16 = jnp.abs(vd16)
        vdir16 = vd16 / dn16
        ii16 = jnp.where(s_nan, zeros16, jnp.abs(STD * vdir16))
        jj16 = jnp.abs(0.0 * vdir16)
        viou16 = (ii16 + jj16) / (dn16 + ii16 + jj16)
        tot = tot + jnp.where(c_pos, viou16, zeros16)
        cnt = cnt + jnp.where(c_pos, ones16, zeros16)

        tot_v = jnp.full((16,), jnp.sum(tot), _F32)
        cnt_v = jnp.full((16,), jnp.sum(cnt), _F32)
        outv[...] = tot_v / cnt_v
        pltpu.sync_copy(outv.at[pl.ds(0, 1)], out_hbm)


@jax.jit
def _idloss(pred_id, target_id):
    mesh = plsc.VectorSubcoreMesh(core_axis_name="c", subcore_axis_name="s",
                                  num_cores=1)
    out = pl.kernel(
        _idloss_body,
        mesh=mesh,
        compiler_params=pltpu.CompilerParams(needs_layout_passes=False),
        out_type=jax.ShapeDtypeStruct((1,), _F32),
        scratch_types=[
            pltpu.VMEM((_CHUNK,), _F32),
            pltpu.VMEM((_CHUNK,), _I32),
            pltpu.VMEM((16,), _F32),
            pltpu.VMEM((16,), _I32),
            pltpu.VMEM((256,), _F32),
            pltpu.VMEM((256,), _F32),
            pltpu.VMEM((256,), _F32),
            pltpu.VMEM((768,), _F32),
            pltpu.VMEM((48,), _F32),
            pltpu.VMEM((16,), _F32),
            pltpu.VMEM_SHARED((768,), _F32),
            pltpu.SemaphoreType.DMA,
            pltpu.SemaphoreType.DMA,
            pltpu.SemaphoreType.DMA,
        ],
    )(pred_id, target_id)
    return jnp.reshape(out, ())


def kernel(pred_id, target_id):
    return _idloss(pred_id, target_id)
